# baseline (speedup 1.0000x reference)
import functools

import jax
import jax.numpy as jnp
from jax.experimental import pallas as pl
from jax.experimental.pallas import tpu as pltpu
from jax.experimental.pallas import tpu_sc as plsc


def _score_block(adj_ref, x_ref, w1_ref, b1_ref, w2t_ref, b2_ref, out_ref,
                 adj_out_ref, *, bm):
    i = pl.program_id(2)
    a = adj_ref[0, 0]
    adj_out_ref[0, 0] = a
    xf = x_ref[0]
    xb = x_ref[0, pl.ds(i * bm, bm), :]
    ax = jnp.dot(a.astype(jnp.bfloat16), xf.astype(jnp.bfloat16),
                 preferred_element_type=jnp.float32)
    denom = jnp.sum(a, axis=1, keepdims=True) + 1.0
    w1 = w1_ref[...].astype(jnp.bfloat16)
    b1 = b1_ref[...]
    axw = (jnp.dot(ax.astype(jnp.bfloat16), w1,
                   preferred_element_type=jnp.float32) + b1) + (
        jnp.dot(xb.astype(jnp.bfloat16), w1,
                preferred_element_type=jnp.float32) + b1)
    axw = axw / denom
    g = jnp.maximum(axw, 0.0)
    gb = g.astype(jnp.bfloat16).astype(jnp.float32)
    w2 = w2t_ref[...].astype(jnp.bfloat16).astype(jnp.float32)
    score = jnp.sum(gb * w2, axis=1, keepdims=True) + b2_ref[0, 0]
    out_ref[0, 0] = score


def _sc_select_body(scores_hbm, sel_hbm, row_v, keys_v, sel_v, acc_v,
                    *, k, rows, n):
    info = plsc.get_sparse_core_info()
    nc = info.num_cores
    wid = jax.lax.axis_index("s") * nc + jax.lax.axis_index("c")
    nch = n // 16
    int_min = jnp.int32(-2147483648)
    kk = jnp.int32(k)

    @pl.when(wid < rows)
    def _():
        pltpu.sync_copy(scores_hbm.at[pl.ds(wid * n, n)], row_v)

        def to_key(c, _):
            v = row_v[pl.ds(c * 16, 16)]
            ki = jax.lax.bitcast_convert_type(v, jnp.int32)
            ki = ki ^ jax.lax.shift_right_arithmetic(ki, 31) & jnp.int32(
                0x7FFFFFFF)
            keys_v[pl.ds(c * 16, 16)] = ki
            return 0

        jax.lax.fori_loop(0, nch, to_key, 0)

        lane = jax.lax.iota(jnp.int32, 16)

        def count_pass(pred):
            acc_v[...] = jnp.zeros((16,), jnp.int32)

            def body(c, _):
                ki = keys_v[pl.ds(c * 16, 16)]
                acc_v[...] = acc_v[...] + jnp.where(pred(c, ki), 1, 0).astype(
                    jnp.int32)
                return 0

            jax.lax.fori_loop(0, nch, body, 0)
            v = acc_v[...]
            s = v[0]
            for i in range(1, 16):
                s = s + v[i]
            return s

        def count_below(cand):
            return count_pass(lambda c, ki: ki < cand)

        t0 = jnp.where(count_below(jnp.int32(0)) < kk, jnp.int32(0), int_min)

        def bit_round(i, t):
            cand = t + jax.lax.shift_left(jnp.int32(1), jnp.int32(30) - i)
            return jnp.where(count_below(cand) < kk, cand, t)

        t = jax.lax.fori_loop(0, 31, bit_round, t0)
        need = kk - count_below(t)

        def count_tie(bound):
            return count_pass(
                lambda c, ki: (ki == t) & ((c * 16 + lane) < bound))

        def tie_round(i, istar):
            cand = istar + jax.lax.shift_left(jnp.int32(1), jnp.int32(11) - i)
            return jnp.where(count_tie(cand) <= need, cand, istar)

        istar = jax.lax.fori_loop(0, 12, tie_round, jnp.int32(0))

        def mark(c, _):
            ki = keys_v[pl.ds(c * 16, 16)]
            sel = (ki < t) | ((ki == t) & ((c * 16 + lane) < istar))
            sel_v[pl.ds(c * 16, 16)] = jnp.where(sel, 1, 0).astype(jnp.int32)
            return 0

        jax.lax.fori_loop(0, nch, mark, 0)
        pltpu.sync_copy(sel_v, sel_hbm.at[pl.ds(wid * n, n)])


def _union_block(sel_ref, srcmask_ref, out_ref, *, heads, b):
    sel = sel_ref[...]
    union = sel[0:b, :]
    for h in range(1, heads):
        union = union | sel[h * b:(h + 1) * b, :]
    out_ref[...] = jnp.where(union > 0, 0, srcmask_ref[...])


def _select_block(scores_ref, srcmask_ref, out_ref, *, k, heads, b, n):
    s = scores_ref[...]
    key = jax.lax.bitcast_convert_type(s, jnp.int32)
    key = key ^ jax.lax.shift_right_arithmetic(key, 31) & jnp.int32(0x7FFFFFFF)
    rows = heads * b
    int_min = jnp.int32(-2147483648)
    cnt = jnp.sum((key < 0).astype(jnp.int32), axis=1, keepdims=True)
    t = jnp.where(cnt < k, jnp.zeros((rows, 1), jnp.int32),
                  jnp.full((rows, 1), int_min))
    for bit in range(30, -1, -1):
        cand = t + jnp.int32(1 << bit)
        cnt = jnp.sum((key < cand).astype(jnp.int32), axis=1, keepdims=True)
        t = jnp.where(cnt < k, cand, t)
    lt = key < t
    cnt_lt = jnp.sum(lt.astype(jnp.int32), axis=1, keepdims=True)
    need = k - cnt_lt
    eq = (key == t).astype(jnp.int32)
    col = jax.lax.broadcasted_iota(jnp.int32, (rows, n), 1)
    incl = eq
    sh = 1
    while sh < n:
        incl = incl + jnp.where(col >= sh, jnp.roll(incl, sh, axis=1), 0)
        sh *= 2
    prefix_excl = incl - eq
    sel = lt | ((eq > 0) & (prefix_excl < need))
    sel = sel.astype(jnp.int32)
    union = sel[0:b, :]
    for h in range(1, heads):
        union = union | sel[h * b:(h + 1) * b, :]
    out_ref[...] = jnp.where(union > 0, 0, srcmask_ref[...])


def _xcat_block(x_ref, wt_ref, bt_ref, out_ref, *, heads):
    y = jnp.dot(x_ref[0], wt_ref[...],
                preferred_element_type=jnp.float32) + bt_ref[...]
    out_ref[0] = jnp.concatenate([y] * heads, axis=1)


def kernel(adj_list, x, src_mask, W1, b1, W2, b2, Wt, bt):
    heads, b, n, _ = adj_list.shape
    d = x.shape[-1]
    k = int(0.5 * n) + 1
    bm = min(1024, n)
    nb = n // bm

    scores, adj_out = pl.pallas_call(
        functools.partial(_score_block, bm=bm),
        grid=(heads, b, nb),
        in_specs=[
            pl.BlockSpec((1, 1, bm, n), lambda h, bb, i: (h, bb, i, 0)),
            pl.BlockSpec((1, n, d), lambda h, bb, i: (bb, 0, 0)),
            pl.BlockSpec((d, d), lambda h, bb, i: (0, 0)),
            pl.BlockSpec((1, d), lambda h, bb, i: (0, 0)),
            pl.BlockSpec((1, d), lambda h, bb, i: (0, 0)),
            pl.BlockSpec((1, 1), lambda h, bb, i: (0, 0)),
        ],
        out_specs=[
            pl.BlockSpec((1, 1, bm, 1), lambda h, bb, i: (h, bb, i, 0)),
            pl.BlockSpec((1, 1, bm, n), lambda h, bb, i: (h, bb, i, 0)),
        ],
        out_shape=[
            jax.ShapeDtypeStruct((heads, b, nb * bm, 1), jnp.float32),
            jax.ShapeDtypeStruct((heads, b, n, n), jnp.float32),
        ],
        compiler_params=pltpu.CompilerParams(
            dimension_semantics=("parallel", "parallel", "arbitrary")),
    )(adj_list, x, W1, b1.reshape(1, d), W2.reshape(1, d)[:, :],
      b2.reshape(1, 1))

    scores2d = scores.reshape(heads * b, n)
    src2d = src_mask.reshape(b, n).astype(jnp.int32)

    sel_flat = pl.kernel(
        functools.partial(_sc_select_body, k=k, rows=heads * b, n=n),
        out_type=jax.ShapeDtypeStruct((heads * b * n,), jnp.int32),
        mesh=plsc.VectorSubcoreMesh(core_axis_name="c", subcore_axis_name="s"),
        scratch_types=[
            pltpu.VMEM((n,), jnp.float32),
            pltpu.VMEM((n,), jnp.int32),
            pltpu.VMEM((n,), jnp.int32),
            pltpu.VMEM((16,), jnp.int32),
        ],
    )(scores2d.reshape(heads * b * n))
    sel = sel_flat.reshape(heads * b, n)

    mask2d = pl.pallas_call(
        functools.partial(_union_block, heads=heads, b=b),
        out_shape=jax.ShapeDtypeStruct((b, n), jnp.int32),
    )(sel, src2d)
    mask_out = mask2d.astype(jnp.bool_).reshape(b, 1, n)

    x_cat = pl.pallas_call(
        functools.partial(_xcat_block, heads=heads),
        grid=(b,),
        in_specs=[
            pl.BlockSpec((1, n, d), lambda bb: (bb, 0, 0)),
            pl.BlockSpec((d, d // heads), lambda bb: (0, 0)),
            pl.BlockSpec((1, d // heads), lambda bb: (0, 0)),
        ],
        out_specs=pl.BlockSpec((1, n, d), lambda bb: (bb, 0, 0)),
        out_shape=jax.ShapeDtypeStruct((b, n, d), jnp.float32),
    )(x, Wt, bt.reshape(1, d // heads))

    return (adj_out, x_cat, mask_out)


# speedup vs baseline: 1.0324x; 1.0324x over previous
import functools

import jax
import jax.numpy as jnp
from jax.experimental import pallas as pl
from jax.experimental.pallas import tpu as pltpu
from jax.experimental.pallas import tpu_sc as plsc


def _score_block(adj_ref, x_ref, w1_ref, b1_ref, w2t_ref, b2_ref, out_ref,
                 adj_out_ref, *, bm):
    i = pl.program_id(2)
    a = adj_ref[0, 0]
    adj_out_ref[0, 0] = a
    xf = x_ref[0]
    xb = x_ref[0, pl.ds(i * bm, bm), :]
    ax = jnp.dot(a.astype(jnp.bfloat16), xf.astype(jnp.bfloat16),
                 preferred_element_type=jnp.float32)
    denom = jnp.sum(a, axis=1, keepdims=True) + 1.0
    w1 = w1_ref[...].astype(jnp.bfloat16)
    b1 = b1_ref[...]
    axw = (jnp.dot(ax.astype(jnp.bfloat16), w1,
                   preferred_element_type=jnp.float32) + b1) + (
        jnp.dot(xb.astype(jnp.bfloat16), w1,
                preferred_element_type=jnp.float32) + b1)
    axw = axw / denom
    g = jnp.maximum(axw, 0.0)
    gb = g.astype(jnp.bfloat16).astype(jnp.float32)
    w2 = w2t_ref[...].astype(jnp.bfloat16).astype(jnp.float32)
    score = jnp.sum(gb * w2, axis=1, keepdims=True) + b2_ref[0, 0]
    out_ref[0, 0] = score


def _sc_select_body(scores_hbm, sel_hbm, row_v, keys_v, sel_v,
                    acc1_v, acc2_v, acc3_v, *, k, rows, n):
    info = plsc.get_sparse_core_info()
    nc = info.num_cores
    wid = jax.lax.axis_index("s") * nc + jax.lax.axis_index("c")
    nch = n // 16
    int_min = jnp.int32(-2147483648)
    kk = jnp.int32(k)

    @pl.when(wid < rows)
    def _():
        pltpu.sync_copy(scores_hbm.at[pl.ds(wid * n, n)], row_v)

        def to_key(c, _):
            v = row_v[pl.ds(c * 16, 16)]
            ki = jax.lax.bitcast_convert_type(v, jnp.int32)
            ki = ki ^ jax.lax.shift_right_arithmetic(ki, 31) & jnp.int32(
                0x7FFFFFFF)
            keys_v[pl.ds(c * 16, 16)] = ki
            return 0

        jax.lax.fori_loop(0, nch, to_key, 0)

        lane = jax.lax.iota(jnp.int32, 16)

        def fold(acc):
            v = acc[...]
            s = v[0]
            for i in range(1, 16):
                s = s + v[i]
            return s

        def count_pass3(pred1, pred2, pred3):
            z = jnp.zeros((16,), jnp.int32)
            acc1_v[...] = z
            acc2_v[...] = z
            acc3_v[...] = z

            def body(c, _):
                ki = keys_v[pl.ds(c * 16, 16)]
                one = jnp.int32(1)
                zero = jnp.int32(0)
                acc1_v[...] = acc1_v[...] + jnp.where(pred1(c, ki), one, zero)
                acc2_v[...] = acc2_v[...] + jnp.where(pred2(c, ki), one, zero)
                acc3_v[...] = acc3_v[...] + jnp.where(pred3(c, ki), one, zero)
                return 0

            jax.lax.fori_loop(0, nch, body, 0)
            return fold(acc1_v), fold(acc2_v), fold(acc3_v)

        def count_below(cand):
            def body(c, _):
                ki = keys_v[pl.ds(c * 16, 16)]
                acc1_v[...] = acc1_v[...] + jnp.where(ki < cand, 1, 0).astype(
                    jnp.int32)
                return 0

            acc1_v[...] = jnp.zeros((16,), jnp.int32)
            jax.lax.fori_loop(0, nch, body, 0)
            return fold(acc1_v)

        t0 = jnp.where(count_below(jnp.int32(0)) < kk, jnp.int32(0), int_min)

        def bit_round2(i, t):
            b = jnp.int32(29) - 2 * i
            c1 = t + jax.lax.shift_left(jnp.int32(1), b)
            c2 = t + jax.lax.shift_left(jnp.int32(2), b)
            c3 = t + jax.lax.shift_left(jnp.int32(3), b)
            n1, n2, n3 = count_pass3(
                lambda c, ki: ki < c1,
                lambda c, ki: ki < c2,
                lambda c, ki: ki < c3)
            j = (jnp.where(n1 < kk, 1, 0) + jnp.where(n2 < kk, 1, 0)
                 + jnp.where(n3 < kk, 1, 0)).astype(jnp.int32)
            return t + jax.lax.shift_left(j, b)

        t = jax.lax.fori_loop(0, 15, bit_round2, t0)
        t = jnp.where(count_below(t + jnp.int32(1)) < kk,
                      t + jnp.int32(1), t)
        need = kk - count_below(t)

        def tie_round2(i, istar):
            b = jnp.int32(10) - 2 * i
            c1 = istar + jax.lax.shift_left(jnp.int32(1), b)
            c2 = istar + jax.lax.shift_left(jnp.int32(2), b)
            c3 = istar + jax.lax.shift_left(jnp.int32(3), b)
            n1, n2, n3 = count_pass3(
                lambda c, ki: (ki == t) & ((c * 16 + lane) < c1),
                lambda c, ki: (ki == t) & ((c * 16 + lane) < c2),
                lambda c, ki: (ki == t) & ((c * 16 + lane) < c3))
            j = (jnp.where(n1 <= need, 1, 0) + jnp.where(n2 <= need, 1, 0)
                 + jnp.where(n3 <= need, 1, 0)).astype(jnp.int32)
            return istar + jax.lax.shift_left(j, b)

        istar = jax.lax.fori_loop(0, 6, tie_round2, jnp.int32(0))

        def mark(c, _):
            ki = keys_v[pl.ds(c * 16, 16)]
            sel = (ki < t) | ((ki == t) & ((c * 16 + lane) < istar))
            sel_v[pl.ds(c * 16, 16)] = jnp.where(sel, 1, 0).astype(jnp.int32)
            return 0

        jax.lax.fori_loop(0, nch, mark, 0)
        pltpu.sync_copy(sel_v, sel_hbm.at[pl.ds(wid * n, n)])


def _union_block(sel_ref, srcmask_ref, out_ref, *, heads, b):
    sel = sel_ref[...]
    union = sel[0:b, :]
    for h in range(1, heads):
        union = union | sel[h * b:(h + 1) * b, :]
    out_ref[...] = jnp.where(union > 0, 0, srcmask_ref[...])


def _select_block(scores_ref, srcmask_ref, out_ref, *, k, heads, b, n):
    s = scores_ref[...]
    key = jax.lax.bitcast_convert_type(s, jnp.int32)
    key = key ^ jax.lax.shift_right_arithmetic(key, 31) & jnp.int32(0x7FFFFFFF)
    rows = heads * b
    int_min = jnp.int32(-2147483648)
    cnt = jnp.sum((key < 0).astype(jnp.int32), axis=1, keepdims=True)
    t = jnp.where(cnt < k, jnp.zeros((rows, 1), jnp.int32),
                  jnp.full((rows, 1), int_min))
    for bit in range(30, -1, -1):
        cand = t + jnp.int32(1 << bit)
        cnt = jnp.sum((key < cand).astype(jnp.int32), axis=1, keepdims=True)
        t = jnp.where(cnt < k, cand, t)
    lt = key < t
    cnt_lt = jnp.sum(lt.astype(jnp.int32), axis=1, keepdims=True)
    need = k - cnt_lt
    eq = (key == t).astype(jnp.int32)
    col = jax.lax.broadcasted_iota(jnp.int32, (rows, n), 1)
    incl = eq
    sh = 1
    while sh < n:
        incl = incl + jnp.where(col >= sh, jnp.roll(incl, sh, axis=1), 0)
        sh *= 2
    prefix_excl = incl - eq
    sel = lt | ((eq > 0) & (prefix_excl < need))
    sel = sel.astype(jnp.int32)
    union = sel[0:b, :]
    for h in range(1, heads):
        union = union | sel[h * b:(h + 1) * b, :]
    out_ref[...] = jnp.where(union > 0, 0, srcmask_ref[...])


def _xcat_block(x_ref, wt_ref, bt_ref, out_ref, *, heads):
    y = jnp.dot(x_ref[0], wt_ref[...],
                preferred_element_type=jnp.float32) + bt_ref[...]
    out_ref[0] = jnp.concatenate([y] * heads, axis=1)


def kernel(adj_list, x, src_mask, W1, b1, W2, b2, Wt, bt):
    heads, b, n, _ = adj_list.shape
    d = x.shape[-1]
    k = int(0.5 * n) + 1
    bm = min(1024, n)
    nb = n // bm

    scores, adj_out = pl.pallas_call(
        functools.partial(_score_block, bm=bm),
        grid=(heads, b, nb),
        in_specs=[
            pl.BlockSpec((1, 1, bm, n), lambda h, bb, i: (h, bb, i, 0)),
            pl.BlockSpec((1, n, d), lambda h, bb, i: (bb, 0, 0)),
            pl.BlockSpec((d, d), lambda h, bb, i: (0, 0)),
            pl.BlockSpec((1, d), lambda h, bb, i: (0, 0)),
            pl.BlockSpec((1, d), lambda h, bb, i: (0, 0)),
            pl.BlockSpec((1, 1), lambda h, bb, i: (0, 0)),
        ],
        out_specs=[
            pl.BlockSpec((1, 1, bm, 1), lambda h, bb, i: (h, bb, i, 0)),
            pl.BlockSpec((1, 1, bm, n), lambda h, bb, i: (h, bb, i, 0)),
        ],
        out_shape=[
            jax.ShapeDtypeStruct((heads, b, nb * bm, 1), jnp.float32),
            jax.ShapeDtypeStruct((heads, b, n, n), jnp.float32),
        ],
        compiler_params=pltpu.CompilerParams(
            dimension_semantics=("parallel", "parallel", "arbitrary")),
    )(adj_list, x, W1, b1.reshape(1, d), W2.reshape(1, d)[:, :],
      b2.reshape(1, 1))

    scores2d = scores.reshape(heads * b, n)
    src2d = src_mask.reshape(b, n).astype(jnp.int32)

    sel_flat = pl.kernel(
        functools.partial(_sc_select_body, k=k, rows=heads * b, n=n),
        out_type=jax.ShapeDtypeStruct((heads * b * n,), jnp.int32),
        mesh=plsc.VectorSubcoreMesh(core_axis_name="c", subcore_axis_name="s"),
        scratch_types=[
            pltpu.VMEM((n,), jnp.float32),
            pltpu.VMEM((n,), jnp.int32),
            pltpu.VMEM((n,), jnp.int32),
            pltpu.VMEM((16,), jnp.int32),
            pltpu.VMEM((16,), jnp.int32),
            pltpu.VMEM((16,), jnp.int32),
        ],
    )(scores2d.reshape(heads * b * n))
    sel = sel_flat.reshape(heads * b, n)

    mask2d = pl.pallas_call(
        functools.partial(_union_block, heads=heads, b=b),
        out_shape=jax.ShapeDtypeStruct((b, n), jnp.int32),
    )(sel, src2d)
    mask_out = mask2d.astype(jnp.bool_).reshape(b, 1, n)

    x_cat = pl.pallas_call(
        functools.partial(_xcat_block, heads=heads),
        grid=(b,),
        in_specs=[
            pl.BlockSpec((1, n, d), lambda bb: (bb, 0, 0)),
            pl.BlockSpec((d, d // heads), lambda bb: (0, 0)),
            pl.BlockSpec((1, d // heads), lambda bb: (0, 0)),
        ],
        out_specs=pl.BlockSpec((1, n, d), lambda bb: (bb, 0, 0)),
        out_shape=jax.ShapeDtypeStruct((b, n, d), jnp.float32),
    )(x, Wt, bt.reshape(1, d // heads))

    return (adj_out, x_cat, mask_out)


# speedup vs baseline: 1.0401x; 1.0075x over previous
import functools

import jax
import jax.numpy as jnp
from jax.experimental import pallas as pl
from jax.experimental.pallas import tpu as pltpu
from jax.experimental.pallas import tpu_sc as plsc


def _score_block(adj_ref, x_ref, w1_ref, b1_ref, w2t_ref, b2_ref, out_ref,
                 adj_out_ref, *, bm):
    i = pl.program_id(2)
    a = adj_ref[0, 0]
    adj_out_ref[0, 0] = a
    xf = x_ref[0]
    xb = x_ref[0, pl.ds(i * bm, bm), :]
    ax = jnp.dot(a.astype(jnp.bfloat16), xf.astype(jnp.bfloat16),
                 preferred_element_type=jnp.float32)
    denom = jnp.sum(a, axis=1, keepdims=True) + 1.0
    w1 = w1_ref[...].astype(jnp.bfloat16)
    b1 = b1_ref[...]
    axw = (jnp.dot(ax.astype(jnp.bfloat16), w1,
                   preferred_element_type=jnp.float32) + b1) + (
        jnp.dot(xb.astype(jnp.bfloat16), w1,
                preferred_element_type=jnp.float32) + b1)
    axw = axw / denom
    g = jnp.maximum(axw, 0.0)
    gb = g.astype(jnp.bfloat16).astype(jnp.float32)
    w2 = w2t_ref[...].astype(jnp.bfloat16).astype(jnp.float32)
    score = jnp.sum(gb * w2, axis=1, keepdims=True) + b2_ref[0, 0]
    out_ref[0, 0] = score


def _sc_select_body(scores_hbm, sel_hbm, row_v, keys_v, sel_v,
                    acc1_v, acc2_v, acc3_v, *, k, rows, n):
    info = plsc.get_sparse_core_info()
    nc = info.num_cores
    wid = jax.lax.axis_index("s") * nc + jax.lax.axis_index("c")
    nch = n // 16
    int_min = jnp.int32(-2147483648)
    kk = jnp.int32(k)

    @pl.when(wid < rows)
    def _():
        pltpu.sync_copy(scores_hbm.at[pl.ds(wid * n, n)], row_v)

        def to_key(c, _):
            v = row_v[pl.ds(c * 16, 16)]
            ki = jax.lax.bitcast_convert_type(v, jnp.int32)
            ki = ki ^ jax.lax.shift_right_arithmetic(ki, 31) & jnp.int32(
                0x7FFFFFFF)
            keys_v[pl.ds(c * 16, 16)] = ki
            return 0

        jax.lax.fori_loop(0, nch, to_key, 0)

        lane = jax.lax.iota(jnp.int32, 16)

        def fold(v):
            s = v[0]
            for i in range(1, 16):
                s = s + v[i]
            return s

        def count_pass3(pred1, pred2, pred3):
            z = jnp.zeros((16,), jnp.int32)
            one = jnp.int32(1)
            zero = jnp.int32(0)

            def body(c2, accs):
                a1, a2, a3 = accs
                for u in range(2):
                    c = 2 * c2 + u
                    ki = keys_v[pl.ds(c * 16, 16)]
                    a1 = a1 + jnp.where(pred1(c, ki), one, zero)
                    a2 = a2 + jnp.where(pred2(c, ki), one, zero)
                    a3 = a3 + jnp.where(pred3(c, ki), one, zero)
                return (a1, a2, a3)

            a1, a2, a3 = jax.lax.fori_loop(0, nch // 2, body, (z, z, z))
            return fold(a1), fold(a2), fold(a3)

        def count_below(cand):
            def body(c2, acc):
                for u in range(2):
                    c = 2 * c2 + u
                    ki = keys_v[pl.ds(c * 16, 16)]
                    acc = acc + jnp.where(ki < cand, 1, 0).astype(jnp.int32)
                return acc

            acc = jax.lax.fori_loop(0, nch // 2, body,
                                    jnp.zeros((16,), jnp.int32))
            return fold(acc)

        t0 = jnp.where(count_below(jnp.int32(0)) < kk, jnp.int32(0), int_min)

        def bit_round2(i, t):
            b = jnp.int32(29) - 2 * i
            c1 = t + jax.lax.shift_left(jnp.int32(1), b)
            c2 = t + jax.lax.shift_left(jnp.int32(2), b)
            c3 = t + jax.lax.shift_left(jnp.int32(3), b)
            n1, n2, n3 = count_pass3(
                lambda c, ki: ki < c1,
                lambda c, ki: ki < c2,
                lambda c, ki: ki < c3)
            j = (jnp.where(n1 < kk, 1, 0) + jnp.where(n2 < kk, 1, 0)
                 + jnp.where(n3 < kk, 1, 0)).astype(jnp.int32)
            return t + jax.lax.shift_left(j, b)

        t = jax.lax.fori_loop(0, 15, bit_round2, t0)
        t = jnp.where(count_below(t + jnp.int32(1)) < kk,
                      t + jnp.int32(1), t)
        need = kk - count_below(t)

        def tie_round2(i, istar):
            b = jnp.int32(10) - 2 * i
            c1 = istar + jax.lax.shift_left(jnp.int32(1), b)
            c2 = istar + jax.lax.shift_left(jnp.int32(2), b)
            c3 = istar + jax.lax.shift_left(jnp.int32(3), b)
            n1, n2, n3 = count_pass3(
                lambda c, ki: (ki == t) & ((c * 16 + lane) < c1),
                lambda c, ki: (ki == t) & ((c * 16 + lane) < c2),
                lambda c, ki: (ki == t) & ((c * 16 + lane) < c3))
            j = (jnp.where(n1 <= need, 1, 0) + jnp.where(n2 <= need, 1, 0)
                 + jnp.where(n3 <= need, 1, 0)).astype(jnp.int32)
            return istar + jax.lax.shift_left(j, b)

        istar = jax.lax.fori_loop(0, 6, tie_round2, jnp.int32(0))

        def mark(c, _):
            ki = keys_v[pl.ds(c * 16, 16)]
            sel = (ki < t) | ((ki == t) & ((c * 16 + lane) < istar))
            sel_v[pl.ds(c * 16, 16)] = jnp.where(sel, 1, 0).astype(jnp.int32)
            return 0

        jax.lax.fori_loop(0, nch, mark, 0)
        pltpu.sync_copy(sel_v, sel_hbm.at[pl.ds(wid * n, n)])


def _union_block(sel_ref, srcmask_ref, out_ref, *, heads, b):
    sel = sel_ref[...]
    union = sel[0:b, :]
    for h in range(1, heads):
        union = union | sel[h * b:(h + 1) * b, :]
    out_ref[...] = jnp.where(union > 0, 0, srcmask_ref[...])


def _select_block(scores_ref, srcmask_ref, out_ref, *, k, heads, b, n):
    s = scores_ref[...]
    key = jax.lax.bitcast_convert_type(s, jnp.int32)
    key = key ^ jax.lax.shift_right_arithmetic(key, 31) & jnp.int32(0x7FFFFFFF)
    rows = heads * b
    int_min = jnp.int32(-2147483648)
    cnt = jnp.sum((key < 0).astype(jnp.int32), axis=1, keepdims=True)
    t = jnp.where(cnt < k, jnp.zeros((rows, 1), jnp.int32),
                  jnp.full((rows, 1), int_min))
    for bit in range(30, -1, -1):
        cand = t + jnp.int32(1 << bit)
        cnt = jnp.sum((key < cand).astype(jnp.int32), axis=1, keepdims=True)
        t = jnp.where(cnt < k, cand, t)
    lt = key < t
    cnt_lt = jnp.sum(lt.astype(jnp.int32), axis=1, keepdims=True)
    need = k - cnt_lt
    eq = (key == t).astype(jnp.int32)
    col = jax.lax.broadcasted_iota(jnp.int32, (rows, n), 1)
    incl = eq
    sh = 1
    while sh < n:
        incl = incl + jnp.where(col >= sh, jnp.roll(incl, sh, axis=1), 0)
        sh *= 2
    prefix_excl = incl - eq
    sel = lt | ((eq > 0) & (prefix_excl < need))
    sel = sel.astype(jnp.int32)
    union = sel[0:b, :]
    for h in range(1, heads):
        union = union | sel[h * b:(h + 1) * b, :]
    out_ref[...] = jnp.where(union > 0, 0, srcmask_ref[...])


def _xcat_block(x_ref, wt_ref, bt_ref, out_ref, *, heads):
    y = jnp.dot(x_ref[0], wt_ref[...],
                preferred_element_type=jnp.float32) + bt_ref[...]
    out_ref[0] = jnp.concatenate([y] * heads, axis=1)


def kernel(adj_list, x, src_mask, W1, b1, W2, b2, Wt, bt):
    heads, b, n, _ = adj_list.shape
    d = x.shape[-1]
    k = int(0.5 * n) + 1
    bm = min(1024, n)
    nb = n // bm

    scores, adj_out = pl.pallas_call(
        functools.partial(_score_block, bm=bm),
        grid=(heads, b, nb),
        in_specs=[
            pl.BlockSpec((1, 1, bm, n), lambda h, bb, i: (h, bb, i, 0)),
            pl.BlockSpec((1, n, d), lambda h, bb, i: (bb, 0, 0)),
            pl.BlockSpec((d, d), lambda h, bb, i: (0, 0)),
            pl.BlockSpec((1, d), lambda h, bb, i: (0, 0)),
            pl.BlockSpec((1, d), lambda h, bb, i: (0, 0)),
            pl.BlockSpec((1, 1), lambda h, bb, i: (0, 0)),
        ],
        out_specs=[
            pl.BlockSpec((1, 1, bm, 1), lambda h, bb, i: (h, bb, i, 0)),
            pl.BlockSpec((1, 1, bm, n), lambda h, bb, i: (h, bb, i, 0)),
        ],
        out_shape=[
            jax.ShapeDtypeStruct((heads, b, nb * bm, 1), jnp.float32),
            jax.ShapeDtypeStruct((heads, b, n, n), jnp.float32),
        ],
        compiler_params=pltpu.CompilerParams(
            dimension_semantics=("parallel", "parallel", "arbitrary")),
    )(adj_list, x, W1, b1.reshape(1, d), W2.reshape(1, d)[:, :],
      b2.reshape(1, 1))

    scores2d = scores.reshape(heads * b, n)
    src2d = src_mask.reshape(b, n).astype(jnp.int32)

    sel_flat = pl.kernel(
        functools.partial(_sc_select_body, k=k, rows=heads * b, n=n),
        out_type=jax.ShapeDtypeStruct((heads * b * n,), jnp.int32),
        mesh=plsc.VectorSubcoreMesh(core_axis_name="c", subcore_axis_name="s"),
        scratch_types=[
            pltpu.VMEM((n,), jnp.float32),
            pltpu.VMEM((n,), jnp.int32),
            pltpu.VMEM((n,), jnp.int32),
            pltpu.VMEM((16,), jnp.int32),
            pltpu.VMEM((16,), jnp.int32),
            pltpu.VMEM((16,), jnp.int32),
        ],
    )(scores2d.reshape(heads * b * n))
    sel = sel_flat.reshape(heads * b, n)

    mask2d = pl.pallas_call(
        functools.partial(_union_block, heads=heads, b=b),
        out_shape=jax.ShapeDtypeStruct((b, n), jnp.int32),
    )(sel, src2d)
    mask_out = mask2d.astype(jnp.bool_).reshape(b, 1, n)

    x_cat = pl.pallas_call(
        functools.partial(_xcat_block, heads=heads),
        grid=(b,),
        in_specs=[
            pl.BlockSpec((1, n, d), lambda bb: (bb, 0, 0)),
            pl.BlockSpec((d, d // heads), lambda bb: (0, 0)),
            pl.BlockSpec((1, d // heads), lambda bb: (0, 0)),
        ],
        out_specs=pl.BlockSpec((1, n, d), lambda bb: (bb, 0, 0)),
        out_shape=jax.ShapeDtypeStruct((b, n, d), jnp.float32),
    )(x, Wt, bt.reshape(1, d // heads))

    return (adj_out, x_cat, mask_out)


# speedup vs baseline: 1.0404x; 1.0003x over previous
import functools

import jax
import jax.numpy as jnp
from jax.experimental import pallas as pl
from jax.experimental.pallas import tpu as pltpu
from jax.experimental.pallas import tpu_sc as plsc


def _score_block(adj_ref, x_ref, w1_ref, b1_ref, w2t_ref, b2_ref, out_ref,
                 adj_out_ref, *, bm):
    i = pl.program_id(2)
    a = adj_ref[0, 0]
    adj_out_ref[0, 0] = a
    xf = x_ref[0]
    xb = x_ref[0, pl.ds(i * bm, bm), :]
    ax = jnp.dot(a.astype(jnp.bfloat16), xf.astype(jnp.bfloat16),
                 preferred_element_type=jnp.float32)
    denom = jnp.sum(a, axis=1, keepdims=True) + 1.0
    w1 = w1_ref[...].astype(jnp.bfloat16)
    b1 = b1_ref[...]
    axw = (jnp.dot(ax.astype(jnp.bfloat16), w1,
                   preferred_element_type=jnp.float32) + b1) + (
        jnp.dot(xb.astype(jnp.bfloat16), w1,
                preferred_element_type=jnp.float32) + b1)
    axw = axw / denom
    g = jnp.maximum(axw, 0.0)
    gb = g.astype(jnp.bfloat16).astype(jnp.float32)
    w2 = w2t_ref[...].astype(jnp.bfloat16).astype(jnp.float32)
    score = jnp.sum(gb * w2, axis=1, keepdims=True) + b2_ref[0, 0]
    out_ref[0, 0] = score


def _sc_select_body(scores_hbm, sel_hbm, row_v, keys_v, sel_v,
                    *, k, rows, n):
    info = plsc.get_sparse_core_info()
    nc = info.num_cores
    wid = jax.lax.axis_index("s") * nc + jax.lax.axis_index("c")
    nch = n // 16
    int_min = jnp.int32(-2147483648)
    kk = jnp.int32(k)

    @pl.when(wid < rows)
    def _():
        pltpu.sync_copy(scores_hbm.at[pl.ds(wid * n, n)], row_v)

        lane = jax.lax.iota(jnp.int32, 16)

        def fold(v):
            s = v[0]
            for i in range(1, 16):
                s = s + v[i]
            return s

        def count_pass3(pred1, pred2, pred3):
            z = jnp.zeros((16,), jnp.int32)
            one = jnp.int32(1)
            zero = jnp.int32(0)

            def body(c2, accs):
                a1, a2, a3 = accs
                for u in range(2):
                    c = 2 * c2 + u
                    ki = keys_v[pl.ds(c * 16, 16)]
                    a1 = a1 + jnp.where(pred1(c, ki), one, zero)
                    a2 = a2 + jnp.where(pred2(c, ki), one, zero)
                    a3 = a3 + jnp.where(pred3(c, ki), one, zero)
                return (a1, a2, a3)

            a1, a2, a3 = jax.lax.fori_loop(0, nch // 2, body, (z, z, z))
            return fold(a1), fold(a2), fold(a3)

        def to_key(c2, acc):
            for u in range(2):
                c = 2 * c2 + u
                v = row_v[pl.ds(c * 16, 16)]
                ki = jax.lax.bitcast_convert_type(v, jnp.int32)
                ki = ki ^ jax.lax.shift_right_arithmetic(ki, 31) & jnp.int32(
                    0x7FFFFFFF)
                keys_v[pl.ds(c * 16, 16)] = ki
                acc = acc + jnp.where(ki < 0, 1, 0).astype(jnp.int32)
            return acc

        nneg = fold(jax.lax.fori_loop(0, nch // 2, to_key,
                                      jnp.zeros((16,), jnp.int32)))
        t0 = jnp.where(nneg < kk, jnp.int32(0), int_min)

        def bit_round2(i, t):
            b = jnp.int32(29) - 2 * i
            c1 = t + jax.lax.shift_left(jnp.int32(1), b)
            c2 = t + jax.lax.shift_left(jnp.int32(2), b)
            c3 = t + jax.lax.shift_left(jnp.int32(3), b)
            n1, n2, n3 = count_pass3(
                lambda c, ki: ki < c1,
                lambda c, ki: ki < c2,
                lambda c, ki: ki < c3)
            j = (jnp.where(n1 < kk, 1, 0) + jnp.where(n2 < kk, 1, 0)
                 + jnp.where(n3 < kk, 1, 0)).astype(jnp.int32)
            return t + jax.lax.shift_left(j, b)

        t = jax.lax.fori_loop(0, 15, bit_round2, t0)
        def final_body(c2, accs):
            a1, a2, a3 = accs
            for u in range(2):
                c = 2 * c2 + u
                ki = keys_v[pl.ds(c * 16, 16)]
                a1 = a1 + jnp.where(ki < t, 1, 0).astype(jnp.int32)
                a2 = a2 + jnp.where(ki < t + 1, 1, 0).astype(jnp.int32)
                a3 = a3 + jnp.where(ki < t + 2, 1, 0).astype(jnp.int32)
            return (a1, a2, a3)

        z16 = jnp.zeros((16,), jnp.int32)
        a1, a2, a3 = jax.lax.fori_loop(0, nch // 2, final_body,
                                       (z16, z16, z16))
        n_lt_t, n_lt_t1, n_lt_t2 = fold(a1), fold(a2), fold(a3)
        take1 = n_lt_t1 < kk
        t = jnp.where(take1, t + jnp.int32(1), t)
        n_lt = jnp.where(take1, n_lt_t1, n_lt_t)
        n_le = jnp.where(take1, n_lt_t2, n_lt_t1)
        need = kk - n_lt
        n_eq = n_le - n_lt

        def tie_search(_):
            def tie_round2(i, istar):
                b = jnp.int32(10) - 2 * i
                c1 = istar + jax.lax.shift_left(jnp.int32(1), b)
                c2 = istar + jax.lax.shift_left(jnp.int32(2), b)
                c3 = istar + jax.lax.shift_left(jnp.int32(3), b)
                n1, n2, n3 = count_pass3(
                    lambda c, ki: (ki == t) & ((c * 16 + lane) < c1),
                    lambda c, ki: (ki == t) & ((c * 16 + lane) < c2),
                    lambda c, ki: (ki == t) & ((c * 16 + lane) < c3))
                j = (jnp.where(n1 <= need, 1, 0)
                     + jnp.where(n2 <= need, 1, 0)
                     + jnp.where(n3 <= need, 1, 0)).astype(jnp.int32)
                return istar + jax.lax.shift_left(j, b)

            return jax.lax.fori_loop(0, 6, tie_round2, jnp.int32(0))

        istar = jax.lax.cond(n_eq <= need,
                             lambda _: jnp.int32(n), tie_search, 0)

        def mark(c, _):
            ki = keys_v[pl.ds(c * 16, 16)]
            sel = (ki < t) | ((ki == t) & ((c * 16 + lane) < istar))
            sel_v[pl.ds(c * 16, 16)] = jnp.where(sel, 1, 0).astype(jnp.int32)
            return 0

        jax.lax.fori_loop(0, nch, mark, 0)
        pltpu.sync_copy(sel_v, sel_hbm.at[pl.ds(wid * n, n)])


def _union_block(sel_ref, srcmask_ref, out_ref, *, heads, b):
    sel = sel_ref[...]
    union = sel[0:b, :]
    for h in range(1, heads):
        union = union | sel[h * b:(h + 1) * b, :]
    out_ref[...] = jnp.where(union > 0, 0, srcmask_ref[...])


def _select_block(scores_ref, srcmask_ref, out_ref, *, k, heads, b, n):
    s = scores_ref[...]
    key = jax.lax.bitcast_convert_type(s, jnp.int32)
    key = key ^ jax.lax.shift_right_arithmetic(key, 31) & jnp.int32(0x7FFFFFFF)
    rows = heads * b
    int_min = jnp.int32(-2147483648)
    cnt = jnp.sum((key < 0).astype(jnp.int32), axis=1, keepdims=True)
    t = jnp.where(cnt < k, jnp.zeros((rows, 1), jnp.int32),
                  jnp.full((rows, 1), int_min))
    for bit in range(30, -1, -1):
        cand = t + jnp.int32(1 << bit)
        cnt = jnp.sum((key < cand).astype(jnp.int32), axis=1, keepdims=True)
        t = jnp.where(cnt < k, cand, t)
    lt = key < t
    cnt_lt = jnp.sum(lt.astype(jnp.int32), axis=1, keepdims=True)
    need = k - cnt_lt
    eq = (key == t).astype(jnp.int32)
    col = jax.lax.broadcasted_iota(jnp.int32, (rows, n), 1)
    incl = eq
    sh = 1
    while sh < n:
        incl = incl + jnp.where(col >= sh, jnp.roll(incl, sh, axis=1), 0)
        sh *= 2
    prefix_excl = incl - eq
    sel = lt | ((eq > 0) & (prefix_excl < need))
    sel = sel.astype(jnp.int32)
    union = sel[0:b, :]
    for h in range(1, heads):
        union = union | sel[h * b:(h + 1) * b, :]
    out_ref[...] = jnp.where(union > 0, 0, srcmask_ref[...])


def _xcat_block(x_ref, wt_ref, bt_ref, out_ref, *, heads):
    y = jnp.dot(x_ref[0], wt_ref[...],
                preferred_element_type=jnp.float32) + bt_ref[...]
    out_ref[0] = jnp.concatenate([y] * heads, axis=1)


def kernel(adj_list, x, src_mask, W1, b1, W2, b2, Wt, bt):
    heads, b, n, _ = adj_list.shape
    d = x.shape[-1]
    k = int(0.5 * n) + 1
    bm = min(1024, n)
    nb = n // bm

    scores, adj_out = pl.pallas_call(
        functools.partial(_score_block, bm=bm),
        grid=(heads, b, nb),
        in_specs=[
            pl.BlockSpec((1, 1, bm, n), lambda h, bb, i: (h, bb, i, 0)),
            pl.BlockSpec((1, n, d), lambda h, bb, i: (bb, 0, 0)),
            pl.BlockSpec((d, d), lambda h, bb, i: (0, 0)),
            pl.BlockSpec((1, d), lambda h, bb, i: (0, 0)),
            pl.BlockSpec((1, d), lambda h, bb, i: (0, 0)),
            pl.BlockSpec((1, 1), lambda h, bb, i: (0, 0)),
        ],
        out_specs=[
            pl.BlockSpec((1, 1, bm, 1), lambda h, bb, i: (h, bb, i, 0)),
            pl.BlockSpec((1, 1, bm, n), lambda h, bb, i: (h, bb, i, 0)),
        ],
        out_shape=[
            jax.ShapeDtypeStruct((heads, b, nb * bm, 1), jnp.float32),
            jax.ShapeDtypeStruct((heads, b, n, n), jnp.float32),
        ],
        compiler_params=pltpu.CompilerParams(
            dimension_semantics=("parallel", "parallel", "arbitrary")),
    )(adj_list, x, W1, b1.reshape(1, d), W2.reshape(1, d)[:, :],
      b2.reshape(1, 1))

    scores2d = scores.reshape(heads * b, n)
    src2d = src_mask.reshape(b, n).astype(jnp.int32)

    sel_flat = pl.kernel(
        functools.partial(_sc_select_body, k=k, rows=heads * b, n=n),
        out_type=jax.ShapeDtypeStruct((heads * b * n,), jnp.int32),
        mesh=plsc.VectorSubcoreMesh(core_axis_name="c", subcore_axis_name="s"),
        scratch_types=[
            pltpu.VMEM((n,), jnp.float32),
            pltpu.VMEM((n,), jnp.int32),
            pltpu.VMEM((n,), jnp.int32),
        ],
    )(scores2d.reshape(heads * b * n))
    sel = sel_flat.reshape(heads * b, n)

    mask2d = pl.pallas_call(
        functools.partial(_union_block, heads=heads, b=b),
        out_shape=jax.ShapeDtypeStruct((b, n), jnp.int32),
    )(sel, src2d)
    mask_out = mask2d.astype(jnp.bool_).reshape(b, 1, n)

    x_cat = pl.pallas_call(
        functools.partial(_xcat_block, heads=heads),
        grid=(b,),
        in_specs=[
            pl.BlockSpec((1, n, d), lambda bb: (bb, 0, 0)),
            pl.BlockSpec((d, d // heads), lambda bb: (0, 0)),
            pl.BlockSpec((1, d // heads), lambda bb: (0, 0)),
        ],
        out_specs=pl.BlockSpec((1, n, d), lambda bb: (bb, 0, 0)),
        out_shape=jax.ShapeDtypeStruct((b, n, d), jnp.float32),
    )(x, Wt, bt.reshape(1, d // heads))

    return (adj_out, x_cat, mask_out)


# speedup vs baseline: 1.0417x; 1.0012x over previous
import functools

import jax
import jax.numpy as jnp
from jax.experimental import pallas as pl
from jax.experimental.pallas import tpu as pltpu
from jax.experimental.pallas import tpu_sc as plsc


def _score_block(adj_ref, x_ref, w1_ref, b1_ref, w2t_ref, b2_ref, out_ref,
                 adj_out_ref, *, bm):
    i = pl.program_id(2)
    a = adj_ref[0, 0]
    adj_out_ref[0, 0] = a
    xf = x_ref[0]
    xb = x_ref[0, pl.ds(i * bm, bm), :]
    ax = jnp.dot(a.astype(jnp.bfloat16), xf.astype(jnp.bfloat16),
                 preferred_element_type=jnp.float32)
    denom = jnp.sum(a, axis=1, keepdims=True) + 1.0
    w1 = w1_ref[...].astype(jnp.bfloat16)
    b1 = b1_ref[...]
    axw = (jnp.dot(ax.astype(jnp.bfloat16), w1,
                   preferred_element_type=jnp.float32) + b1) + (
        jnp.dot(xb.astype(jnp.bfloat16), w1,
                preferred_element_type=jnp.float32) + b1)
    axw = axw / denom
    g = jnp.maximum(axw, 0.0)
    gb = g.astype(jnp.bfloat16).astype(jnp.float32)
    w2 = w2t_ref[...].astype(jnp.bfloat16).astype(jnp.float32)
    score = jnp.sum(gb * w2, axis=1, keepdims=True) + b2_ref[0, 0]
    out_ref[0, 0] = score


def _sc_select_body(scores_hbm, sel_hbm, row_v, keys_v, sel_v,
                    *, k, rows, n):
    info = plsc.get_sparse_core_info()
    nc = info.num_cores
    wid = jax.lax.axis_index("s") * nc + jax.lax.axis_index("c")
    nch = n // 16
    int_min = jnp.int32(-2147483648)
    kk = jnp.int32(k)

    @pl.when(wid < rows)
    def _():
        pltpu.sync_copy(scores_hbm.at[pl.ds(wid * n, n)], row_v)

        lane = jax.lax.iota(jnp.int32, 16)

        def fold(v):
            s = v[0]
            for i in range(1, 16):
                s = s + v[i]
            return s

        def count_pass3(pred1, pred2, pred3):
            z = jnp.zeros((16,), jnp.int32)
            one = jnp.int32(1)
            zero = jnp.int32(0)

            def body(c2, accs):
                a1, a2, a3 = accs
                for u in range(2):
                    c = 2 * c2 + u
                    ki = keys_v[pl.ds(c * 16, 16)]
                    a1 = a1 + jnp.where(pred1(c, ki), one, zero)
                    a2 = a2 + jnp.where(pred2(c, ki), one, zero)
                    a3 = a3 + jnp.where(pred3(c, ki), one, zero)
                return (a1, a2, a3)

            a1, a2, a3 = jax.lax.fori_loop(0, nch // 2, body, (z, z, z))
            return fold(a1), fold(a2), fold(a3)

        def to_key(c2, acc):
            for u in range(2):
                c = 2 * c2 + u
                v = row_v[pl.ds(c * 16, 16)]
                ki = jax.lax.bitcast_convert_type(v, jnp.int32)
                ki = ki ^ jax.lax.shift_right_arithmetic(ki, 31) & jnp.int32(
                    0x7FFFFFFF)
                keys_v[pl.ds(c * 16, 16)] = ki
                acc = acc + jnp.where(ki < 0, 1, 0).astype(jnp.int32)
            return acc

        nneg = fold(jax.lax.fori_loop(0, nch // 2, to_key,
                                      jnp.zeros((16,), jnp.int32)))
        t0 = jnp.where(nneg < kk, jnp.int32(0), int_min)

        def bit_round2(i, t):
            b = jnp.int32(29) - 2 * i
            c1 = t + jax.lax.shift_left(jnp.int32(1), b)
            c2 = t + jax.lax.shift_left(jnp.int32(2), b)
            c3 = t + jax.lax.shift_left(jnp.int32(3), b)
            n1, n2, n3 = count_pass3(
                lambda c, ki: ki < c1,
                lambda c, ki: ki < c2,
                lambda c, ki: ki < c3)
            j = (jnp.where(n1 < kk, 1, 0) + jnp.where(n2 < kk, 1, 0)
                 + jnp.where(n3 < kk, 1, 0)).astype(jnp.int32)
            return t + jax.lax.shift_left(j, b)

        t = jax.lax.fori_loop(0, 15, bit_round2, t0)
        def final_body(c2, accs):
            a1, a2, a3 = accs
            for u in range(2):
                c = 2 * c2 + u
                ki = keys_v[pl.ds(c * 16, 16)]
                a1 = a1 + jnp.where(ki < t, 1, 0).astype(jnp.int32)
                a2 = a2 + jnp.where(ki < t + 1, 1, 0).astype(jnp.int32)
                a3 = a3 + jnp.where(ki < t + 2, 1, 0).astype(jnp.int32)
            return (a1, a2, a3)

        z16 = jnp.zeros((16,), jnp.int32)
        a1, a2, a3 = jax.lax.fori_loop(0, nch // 2, final_body,
                                       (z16, z16, z16))
        n_lt_t, n_lt_t1, n_lt_t2 = fold(a1), fold(a2), fold(a3)
        take1 = n_lt_t1 < kk
        t = jnp.where(take1, t + jnp.int32(1), t)
        n_lt = jnp.where(take1, n_lt_t1, n_lt_t)
        n_le = jnp.where(take1, n_lt_t2, n_lt_t1)
        need = kk - n_lt
        n_eq = n_le - n_lt

        def tie_search(_):
            def tie_round2(i, istar):
                b = jnp.int32(10) - 2 * i
                c1 = istar + jax.lax.shift_left(jnp.int32(1), b)
                c2 = istar + jax.lax.shift_left(jnp.int32(2), b)
                c3 = istar + jax.lax.shift_left(jnp.int32(3), b)
                n1, n2, n3 = count_pass3(
                    lambda c, ki: (ki == t) & ((c * 16 + lane) < c1),
                    lambda c, ki: (ki == t) & ((c * 16 + lane) < c2),
                    lambda c, ki: (ki == t) & ((c * 16 + lane) < c3))
                j = (jnp.where(n1 <= need, 1, 0)
                     + jnp.where(n2 <= need, 1, 0)
                     + jnp.where(n3 <= need, 1, 0)).astype(jnp.int32)
                return istar + jax.lax.shift_left(j, b)

            return jax.lax.fori_loop(0, 6, tie_round2, jnp.int32(0))

        istar = jax.lax.cond(n_eq <= need,
                             lambda _: jnp.int32(n), tie_search, 0)

        def mark(c, _):
            ki = keys_v[pl.ds(c * 16, 16)]
            sel = (ki < t) | ((ki == t) & ((c * 16 + lane) < istar))
            sel_v[pl.ds(c * 16, 16)] = jnp.where(sel, 1, 0).astype(jnp.int32)
            return 0

        jax.lax.fori_loop(0, nch, mark, 0)
        pltpu.sync_copy(sel_v, sel_hbm.at[pl.ds(wid * n, n)])


def _union_block(sel_ref, srcmask_ref, out_ref, *, heads, b):
    sel = sel_ref[...]
    union = sel[0:b, :]
    for h in range(1, heads):
        union = union | sel[h * b:(h + 1) * b, :]
    out_ref[...] = jnp.where(union > 0, 0, srcmask_ref[...])


def _xcat_block(x_ref, wt_ref, bt_ref, out_ref, *, heads):
    y = jnp.dot(x_ref[0], wt_ref[...],
                preferred_element_type=jnp.float32) + bt_ref[...]
    out_ref[0] = jnp.concatenate([y] * heads, axis=1)


def kernel(adj_list, x, src_mask, W1, b1, W2, b2, Wt, bt):
    heads, b, n, _ = adj_list.shape
    d = x.shape[-1]
    k = int(0.5 * n) + 1
    bm = min(1024, n)
    nb = n // bm

    scores, adj_out = pl.pallas_call(
        functools.partial(_score_block, bm=bm),
        grid=(heads, b, nb),
        in_specs=[
            pl.BlockSpec((1, 1, bm, n), lambda h, bb, i: (h, bb, i, 0)),
            pl.BlockSpec((1, n, d), lambda h, bb, i: (bb, 0, 0)),
            pl.BlockSpec((d, d), lambda h, bb, i: (0, 0)),
            pl.BlockSpec((1, d), lambda h, bb, i: (0, 0)),
            pl.BlockSpec((1, d), lambda h, bb, i: (0, 0)),
            pl.BlockSpec((1, 1), lambda h, bb, i: (0, 0)),
        ],
        out_specs=[
            pl.BlockSpec((1, 1, bm, 1), lambda h, bb, i: (h, bb, i, 0)),
            pl.BlockSpec((1, 1, bm, n), lambda h, bb, i: (h, bb, i, 0)),
        ],
        out_shape=[
            jax.ShapeDtypeStruct((heads, b, nb * bm, 1), jnp.float32),
            jax.ShapeDtypeStruct((heads, b, n, n), jnp.float32),
        ],
        compiler_params=pltpu.CompilerParams(
            dimension_semantics=("parallel", "parallel", "arbitrary")),
    )(adj_list, x, W1, b1.reshape(1, d), W2.reshape(1, d)[:, :],
      b2.reshape(1, 1))

    scores2d = scores.reshape(heads * b, n)
    src2d = src_mask.reshape(b, n).astype(jnp.int32)

    sel_flat = pl.kernel(
        functools.partial(_sc_select_body, k=k, rows=heads * b, n=n),
        out_type=jax.ShapeDtypeStruct((heads * b * n,), jnp.int32),
        mesh=plsc.VectorSubcoreMesh(core_axis_name="c", subcore_axis_name="s"),
        scratch_types=[
            pltpu.VMEM((n,), jnp.float32),
            pltpu.VMEM((n,), jnp.int32),
            pltpu.VMEM((n,), jnp.int32),
        ],
    )(scores2d.reshape(heads * b * n))
    sel = sel_flat.reshape(heads * b, n)

    mask2d = pl.pallas_call(
        functools.partial(_union_block, heads=heads, b=b),
        out_shape=jax.ShapeDtypeStruct((b, n), jnp.int32),
    )(sel, src2d)
    mask_out = mask2d.astype(jnp.bool_).reshape(b, 1, n)

    x_cat = pl.pallas_call(
        functools.partial(_xcat_block, heads=heads),
        grid=(b,),
        in_specs=[
            pl.BlockSpec((1, n, d), lambda bb: (bb, 0, 0)),
            pl.BlockSpec((d, d // heads), lambda bb: (0, 0)),
            pl.BlockSpec((1, d // heads), lambda bb: (0, 0)),
        ],
        out_specs=pl.BlockSpec((1, n, d), lambda bb: (bb, 0, 0)),
        out_shape=jax.ShapeDtypeStruct((b, n, d), jnp.float32),
    )(x, Wt, bt.reshape(1, d // heads))

    return (adj_out, x_cat, mask_out)
